# baseline (speedup 1.0000x reference)
import functools

import jax
import jax.numpy as jnp
from jax import lax
from jax.experimental import pallas as pl
from jax.experimental.pallas import tpu as pltpu
from jax.experimental.pallas import tpu_sc as plsc

NN = 10000
EE = 320000
ETN = 100000
HD = 256
HH = 128
CE = 80
NSUB = 16
NCORE = 2
NP = 10240
ROWS_PER_SUB = NP // NSUB
ECHUNKS = EE // CE
ECH_PER_SUB = ECHUNKS // NSUB
PCHUNKS = ETN // CE
PCH_PER_W = -(-PCHUNKS // (NCORE * NSUB))

_SC_MESH = plsc.VectorSubcoreMesh(core_axis_name="c", subcore_axis_name="s")


def _edge_body(kt, qvt, srch, dsth, zh, out,
               srcv, dstv, srcb, dstb, qvbuf, kbuf, msgbuf, aggsh,
               sem1, sem2):
    c = lax.axis_index("c")
    s = lax.axis_index("s")
    rows0 = s * ROWS_PER_SUB

    pltpu.sync_copy(zh.at[pl.ds(rows0, ROWS_PER_SUB)],
                    aggsh.at[pl.ds(rows0, ROWS_PER_SUB)])
    plsc.subcore_barrier()

    bias = c * NN

    def chunk_body(j, carry):
        base = (j * NSUB + s) * CE
        pltpu.sync_copy(srch.at[pl.ds(base, CE)], srcv)
        pltpu.sync_copy(dsth.at[pl.ds(base, CE)], dstv)
        for jj in range(CE // 16):
            sl = pl.ds(jj * 16, 16)
            srcb[sl] = srcv[sl] + bias
            dstb[sl] = dstv[sl] + bias
        cp1 = pltpu.async_copy(qvt.at[srcb], qvbuf, sem1)
        cp2 = pltpu.async_copy(kt.at[dstb], kbuf, sem2)
        cp1.wait()
        cp2.wait()

        def edge_body(e, carry2):
            for hc in range(HH // 16):
                sl = pl.ds(hc * 16, 16)
                kvec = kbuf[e, sl]
                qvec = qvbuf[e, sl]
                vvec = qvbuf[e, pl.ds(HH + hc * 16, 16)]
                msgbuf[e, sl] = vvec / (1.0 + jnp.exp(-(kvec + qvec)))
            return carry2

        lax.fori_loop(0, CE, edge_body, 0)
        pltpu.sync_copy(msgbuf, aggsh.at[dstv], add=True)
        return carry

    lax.fori_loop(0, ECH_PER_SUB, chunk_body, 0)
    plsc.subcore_barrier()
    pltpu.sync_copy(aggsh.at[pl.ds(rows0, ROWS_PER_SUB)],
                    out.at[pl.ds(c * NP + rows0, ROWS_PER_SUB)])


_edge_sc = pl.kernel(
    _edge_body,
    out_type=jax.ShapeDtypeStruct((NCORE * NP, HH), jnp.float32),
    mesh=_SC_MESH,
    scratch_types=[
        pltpu.VMEM((CE,), jnp.int32),
        pltpu.VMEM((CE,), jnp.int32),
        pltpu.VMEM((CE,), jnp.int32),
        pltpu.VMEM((CE,), jnp.int32),
        pltpu.VMEM((CE, HD), jnp.float32),
        pltpu.VMEM((CE, HH), jnp.float32),
        pltpu.VMEM((CE, HH), jnp.float32),
        pltpu.VMEM_SHARED((NP, HH), jnp.float32),
        pltpu.SemaphoreType.DMA,
        pltpu.SemaphoreType.DMA,
    ],
)


def _pair_body(tab, tsh, tdh, outa, outb, idx1, idx2, buf1, buf2, sem1, sem2):
    c = lax.axis_index("c")
    s = lax.axis_index("s")
    w = s * NCORE + c

    def chunk_body(j, carry):
        chunk = j * (NCORE * NSUB) + w

        @pl.when(chunk < PCHUNKS)
        def _():
            base = chunk * CE
            pltpu.sync_copy(tsh.at[pl.ds(base, CE)], idx1)
            pltpu.sync_copy(tdh.at[pl.ds(base, CE)], idx2)
            for jj in range(CE // 16):
                sl = pl.ds(jj * 16, 16)
                idx2[sl] = idx2[sl] + NN
            cp1 = pltpu.async_copy(tab.at[idx1], buf1, sem1)
            cp2 = pltpu.async_copy(tab.at[idx2], buf2, sem2)
            cp1.wait()
            cp2.wait()
            pltpu.sync_copy(buf1, outa.at[pl.ds(base, CE)])
            pltpu.sync_copy(buf2, outb.at[pl.ds(base, CE)])

        return carry

    lax.fori_loop(0, PCH_PER_W, chunk_body, 0)


_pair_sc = pl.kernel(
    _pair_body,
    out_type=(jax.ShapeDtypeStruct((ETN, HD), jnp.float32),
              jax.ShapeDtypeStruct((ETN, HD), jnp.float32)),
    mesh=_SC_MESH,
    scratch_types=[
        pltpu.VMEM((CE,), jnp.int32),
        pltpu.VMEM((CE,), jnp.int32),
        pltpu.VMEM((CE, HD), jnp.float32),
        pltpu.VMEM((CE, HD), jnp.float32),
        pltpu.SemaphoreType.DMA,
        pltpu.SemaphoreType.DMA,
    ],
)


def _mm_body(x_ref, w_ref, o_ref):
    o_ref[...] = lax.dot_general(
        x_ref[...], w_ref[...], (((1,), (1,)), ((), ())),
        preferred_element_type=jnp.float32)


def _matmul_t(x, w, block_m=2000):
    M, K = x.shape
    H = w.shape[0]
    return pl.pallas_call(
        _mm_body,
        grid=(M // block_m,),
        in_specs=[pl.BlockSpec((block_m, K), lambda i: (i, 0)),
                  pl.BlockSpec((H, K), lambda i: (0, 0))],
        out_specs=pl.BlockSpec((block_m, H), lambda i: (i, 0)),
        out_shape=jax.ShapeDtypeStruct((M, H), jnp.float32),
    )(x, w)


def _post_body(agg_ref, s_ref, cb_ref, g_ref, b_ref, h_ref):
    t = jnp.maximum(agg_ref[...] + s_ref[...] + cb_ref[...], 0.0)
    mu = jnp.mean(t, axis=-1, keepdims=True)
    var = jnp.mean((t - mu) ** 2, axis=-1, keepdims=True)
    h_ref[...] = (t - mu) * lax.rsqrt(var + 1e-5) * g_ref[...] + b_ref[...]


def _post_conv(agg, s, conv_b, ln_g, ln_b, block_m=2000):
    M, H = agg.shape
    vec = pl.BlockSpec((1, H), lambda i: (0, 0))
    return pl.pallas_call(
        _post_body,
        grid=(M // block_m,),
        in_specs=[pl.BlockSpec((block_m, H), lambda i: (i, 0)),
                  pl.BlockSpec((block_m, H), lambda i: (i, 0)),
                  vec, vec, vec],
        out_specs=pl.BlockSpec((block_m, H), lambda i: (i, 0)),
        out_shape=jax.ShapeDtypeStruct((M, H), jnp.float32),
    )(agg, s, conv_b.reshape(1, H), ln_g.reshape(1, H), ln_b.reshape(1, H))


def _jk_body(h1_ref, h2_ref, agg3_ref, s3_ref, cb3_ref,
             w1_ref, w2_ref, w3_ref, bjk_ref, o_ref):
    h3 = agg3_ref[...] + s3_ref[...] + cb3_ref[...]
    acc = lax.dot_general(h1_ref[...], w1_ref[...], (((1,), (1,)), ((), ())),
                          preferred_element_type=jnp.float32)
    acc += lax.dot_general(h2_ref[...], w2_ref[...], (((1,), (1,)), ((), ())),
                           preferred_element_type=jnp.float32)
    acc += lax.dot_general(h3, w3_ref[...], (((1,), (1,)), ((), ())),
                           preferred_element_type=jnp.float32)
    o_ref[...] = acc + bjk_ref[...]


def _jk(h1, h2, agg3, s3, cb3, wjk, bjk, block_m=2000):
    M, H = h1.shape
    w1 = wjk[:, :H]
    w2 = wjk[:, H:2 * H]
    w3 = wjk[:, 2 * H:]
    blk = pl.BlockSpec((block_m, H), lambda i: (i, 0))
    wblk = pl.BlockSpec((H, H), lambda i: (0, 0))
    vec = pl.BlockSpec((1, H), lambda i: (0, 0))
    return pl.pallas_call(
        _jk_body,
        grid=(M // block_m,),
        in_specs=[blk, blk, blk, blk, vec, wblk, wblk, wblk, vec],
        out_specs=blk,
        out_shape=jax.ShapeDtypeStruct((M, H), jnp.float32),
    )(h1, h2, agg3, s3, cb3.reshape(1, H), w1, w2, w3, bjk.reshape(1, H))


def _mlp_body(pa_ref, pb_ref, po_ref, wpo_ref, b1_ref, w2_ref, b2_ref,
              w3_ref, b3_ref, o_ref):
    z1 = pa_ref[...] + pb_ref[...] + b1_ref[...]
    z1 += lax.dot_general(po_ref[...], wpo_ref[...], (((1,), (1,)), ((), ())),
                          preferred_element_type=jnp.float32)
    z1 = jnp.maximum(z1, 0.0)
    z2 = lax.dot_general(z1, w2_ref[...], (((1,), (1,)), ((), ())),
                         preferred_element_type=jnp.float32) + b2_ref[...]
    z2 = jnp.maximum(z2, 0.0)
    z = jnp.sum(z2 * w3_ref[...], axis=-1, keepdims=True) + b3_ref[...]
    o_ref[...] = jax.nn.sigmoid(z)


def _mlp(pa, pb, po, wpo, b1, w2, b2, w3, b3, block_m=2000):
    M, H = pa.shape
    H2 = w2.shape[0]
    return pl.pallas_call(
        _mlp_body,
        grid=(M // block_m,),
        in_specs=[pl.BlockSpec((block_m, H), lambda i: (i, 0)),
                  pl.BlockSpec((block_m, H), lambda i: (i, 0)),
                  pl.BlockSpec((block_m, 4), lambda i: (i, 0)),
                  pl.BlockSpec((H, 4), lambda i: (0, 0)),
                  pl.BlockSpec((1, H), lambda i: (0, 0)),
                  pl.BlockSpec((H2, H), lambda i: (0, 0)),
                  pl.BlockSpec((1, H2), lambda i: (0, 0)),
                  pl.BlockSpec((1, H2), lambda i: (0, 0)),
                  pl.BlockSpec((1, 1), lambda i: (0, 0))],
        out_specs=pl.BlockSpec((block_m, 1), lambda i: (i, 0)),
        out_shape=jax.ShapeDtypeStruct((M, 1), jnp.float32),
    )(pa, pb, po, wpo, b1, w2, b2, w3, b3)


def kernel(target_edge_index, x, embed_edge_index, pitch_score, onset_score,
           params):
    src, dst = embed_edge_index[0], embed_edge_index[1]
    convs = params['convs']
    zeros_half = jnp.zeros((NP, HH), jnp.float32)

    h = x
    hs = []
    agg3 = None
    s3 = None
    for i in range(3):
        p = convs[i]
        wall = jnp.concatenate([p['Wk'], p['Wq'], p['Wv'], p['Ws']], axis=0)
        kqvs = _matmul_t(h, wall)
        k = kqvs[:, :HD]
        q = kqvs[:, HD:2 * HD]
        v = kqvs[:, 2 * HD:3 * HD]
        s = kqvs[:, 3 * HD:]
        kt = jnp.concatenate([k[:, :HH], k[:, HH:]], axis=0)
        qvt = jnp.concatenate(
            [jnp.concatenate([q[:, :HH], v[:, :HH]], axis=1),
             jnp.concatenate([q[:, HH:], v[:, HH:]], axis=1)],
            axis=0)
        aggf = _edge_sc(kt, qvt, src, dst, zeros_half)
        agg = (aggf.reshape(NCORE, NP, HH)[:, :NN]
               .transpose(1, 0, 2).reshape(NN, HD))
        if i != 2:
            h = _post_conv(agg, s, p['b'], params['ln_g'], params['ln_b'])
            hs.append(h)
        else:
            agg3, s3 = agg, s

    hjk = _jk(hs[0], hs[1], agg3, s3, convs[2]['b'],
              params['Wjk'], params['bjk'])

    wa = params['Wp1'][:, :HD]
    wb = params['Wp1'][:, HD:2 * HD]
    wab = jnp.concatenate([wa, wb], axis=0)
    ab = _matmul_t(hjk, wab)
    tab = jnp.concatenate([ab[:, :HD], ab[:, HD:]], axis=0)

    ts, td = target_edge_index[0], target_edge_index[1]
    pa, pb = _pair_sc(tab, ts, td)

    po = jnp.concatenate(
        [pitch_score, onset_score,
         jnp.zeros((ETN, 1), jnp.float32)], axis=1)
    wpo = jnp.concatenate(
        [params['Wp1'][:, 2 * HD:],
         jnp.zeros((HD, 1), jnp.float32)], axis=1)

    return _mlp(pa, pb, po, wpo,
                params['bp1'].reshape(1, HD),
                params['Wp2'],
                params['bp2'].reshape(1, HD // 2),
                params['Wp3'].reshape(1, HD // 2),
                params['bp3'].reshape(1, 1))


# speedup vs baseline: 3.8570x; 3.8570x over previous
import functools

import jax
import jax.numpy as jnp
from jax import lax
from jax.experimental import pallas as pl
from jax.experimental.pallas import tpu as pltpu
from jax.experimental.pallas import tpu_sc as plsc

NN = 10000
EE = 320000
ETN = 100000
HD = 256
HH = 128
CE = 32
CP = 80
NSUB = 16
NCORE = 2
NP = 10240
ROWS_PER_SUB = NP // NSUB
ECH_PER_SUB = (EE // CE) // NSUB
PCHUNKS = ETN // CP
PCH_PER_W = -(-PCHUNKS // (NCORE * NSUB))

_SC_MESH = plsc.VectorSubcoreMesh(core_axis_name="c", subcore_axis_name="s")


NPAIR = ECH_PER_SUB // 2


def _edge_body(kt, qvt, srch, dsth, zh, out,
               srcv0, srcv1, dstv0, dstv1, srcb0, srcb1, dstb0, dstb1,
               dsts0, dsts1, qv0, qv1, kb0, kb1, msg0, msg1, aggsh,
               semidx0, semidx1, semqv0, semqv1, semk0, semk1,
               semsc0, semsc1):
    c = lax.axis_index("c")
    s = lax.axis_index("s")
    rows0 = s * ROWS_PER_SUB
    bias = c * NN

    slots = (
        (srcv0, dstv0, srcb0, dstb0, dsts0, qv0, kb0, msg0,
         semidx0, semqv0, semk0, semsc0),
        (srcv1, dstv1, srcb1, dstb1, dsts1, qv1, kb1, msg1,
         semidx1, semqv1, semk1, semsc1),
    )

    def issue_idx(p, chunk):
        srcv, dstv = slots[p][0], slots[p][1]
        semidx = slots[p][8]
        base = (chunk * NSUB + s) * CE
        pltpu.async_copy(srch.at[pl.ds(base, CE)], srcv, semidx)
        pltpu.async_copy(dsth.at[pl.ds(base, CE)], dstv, semidx)

    def wait_idx(p):
        srcv, dstv = slots[p][0], slots[p][1]
        semidx = slots[p][8]
        pltpu.make_async_copy(srch.at[pl.ds(0, CE)], srcv, semidx).wait()
        pltpu.make_async_copy(dsth.at[pl.ds(0, CE)], dstv, semidx).wait()

    def bias_and_gather(p):
        srcv, dstv, srcb, dstb, dsts, qv, kb = slots[p][:7]
        semqv, semk = slots[p][9], slots[p][10]
        for jj in range(CE // 16):
            sl = pl.ds(jj * 16, 16)
            dv = dstv[sl]
            srcb[sl] = srcv[sl] + bias
            dstb[sl] = dv + bias
            dsts[sl] = dv
        pltpu.async_copy(qvt.at[srcb], qv, semqv)
        pltpu.async_copy(kt.at[dstb], kb, semk)

    def wait_gathers(p):
        srcb, dstb = slots[p][2], slots[p][3]
        qv, kb = slots[p][5], slots[p][6]
        semqv, semk = slots[p][9], slots[p][10]
        pltpu.make_async_copy(qvt.at[srcb], qv, semqv).wait()
        pltpu.make_async_copy(kt.at[dstb], kb, semk).wait()

    def compute(p):
        qv, kb, msg = slots[p][5], slots[p][6], slots[p][7]

        @plsc.parallel_loop(0, CE)
        def _(e):
            for hc in range(HH // 16):
                sl = pl.ds(hc * 16, 16)
                kvec = kb[e, sl]
                qvec = qv[e, sl]
                vvec = qv[e, pl.ds(HH + hc * 16, 16)]
                msg[e, sl] = vvec / (1.0 + jnp.exp(-(kvec + qvec)))

    def issue_scatter(p):
        dsts, msg, semsc = slots[p][4], slots[p][7], slots[p][11]
        pltpu.async_copy(msg, aggsh.at[dsts], semsc, add=True)

    def wait_scatter(p):
        dsts, msg, semsc = slots[p][4], slots[p][7], slots[p][11]
        pltpu.make_async_copy(msg, aggsh.at[dsts], semsc).wait()

    pltpu.sync_copy(zh.at[pl.ds(rows0, ROWS_PER_SUB)],
                    aggsh.at[pl.ds(rows0, ROWS_PER_SUB)])
    plsc.subcore_barrier()

    issue_idx(0, 0)
    issue_idx(1, 1)

    def pair_body(j, carry):
        wait_idx(0)

        @pl.when(j > 0)
        def _():
            wait_scatter(0)

        bias_and_gather(0)
        wait_idx(1)

        @pl.when(j > 0)
        def _():
            wait_scatter(1)

        bias_and_gather(1)

        issue_idx(0, 2 * j + 2)

        @pl.when(j < NPAIR - 1)
        def _():
            issue_idx(1, 2 * j + 3)

        wait_gathers(0)
        compute(0)
        issue_scatter(0)
        wait_gathers(1)
        compute(1)
        issue_scatter(1)
        return carry

    lax.fori_loop(0, NPAIR, pair_body, 0)

    wait_idx(0)
    wait_scatter(0)
    bias_and_gather(0)
    wait_gathers(0)
    compute(0)
    issue_scatter(0)

    wait_scatter(0)
    wait_scatter(1)
    plsc.subcore_barrier()
    pltpu.sync_copy(aggsh.at[pl.ds(rows0, ROWS_PER_SUB)],
                    out.at[pl.ds(c * NP + rows0, ROWS_PER_SUB)])


_edge_sc = pl.kernel(
    _edge_body,
    out_type=jax.ShapeDtypeStruct((NCORE * NP, HH), jnp.float32),
    mesh=_SC_MESH,
    scratch_types=[
        pltpu.VMEM((CE,), jnp.int32),
        pltpu.VMEM((CE,), jnp.int32),
        pltpu.VMEM((CE,), jnp.int32),
        pltpu.VMEM((CE,), jnp.int32),
        pltpu.VMEM((CE,), jnp.int32),
        pltpu.VMEM((CE,), jnp.int32),
        pltpu.VMEM((CE,), jnp.int32),
        pltpu.VMEM((CE,), jnp.int32),
        pltpu.VMEM((CE,), jnp.int32),
        pltpu.VMEM((CE,), jnp.int32),
        pltpu.VMEM((CE, HD), jnp.float32),
        pltpu.VMEM((CE, HD), jnp.float32),
        pltpu.VMEM((CE, HH), jnp.float32),
        pltpu.VMEM((CE, HH), jnp.float32),
        pltpu.VMEM((CE, HH), jnp.float32),
        pltpu.VMEM((CE, HH), jnp.float32),
        pltpu.VMEM_SHARED((NP, HH), jnp.float32),
        pltpu.SemaphoreType.DMA,
        pltpu.SemaphoreType.DMA,
        pltpu.SemaphoreType.DMA,
        pltpu.SemaphoreType.DMA,
        pltpu.SemaphoreType.DMA,
        pltpu.SemaphoreType.DMA,
        pltpu.SemaphoreType.DMA,
        pltpu.SemaphoreType.DMA,
    ],
)


def _pair_body(tab, tsh, tdh, outa, outb, idx1, idx2, buf1, buf2, sem1, sem2):
    c = lax.axis_index("c")
    s = lax.axis_index("s")
    w = s * NCORE + c

    def chunk_body(j, carry):
        chunk = j * (NCORE * NSUB) + w

        @pl.when(chunk < PCHUNKS)
        def _():
            base = chunk * CP
            pltpu.sync_copy(tsh.at[pl.ds(base, CP)], idx1)
            pltpu.sync_copy(tdh.at[pl.ds(base, CP)], idx2)
            for jj in range(CP // 16):
                sl = pl.ds(jj * 16, 16)
                idx2[sl] = idx2[sl] + NN
            cp1 = pltpu.async_copy(tab.at[idx1], buf1, sem1)
            cp2 = pltpu.async_copy(tab.at[idx2], buf2, sem2)
            cp1.wait()
            cp2.wait()
            pltpu.sync_copy(buf1, outa.at[pl.ds(base, CP)])
            pltpu.sync_copy(buf2, outb.at[pl.ds(base, CP)])

        return carry

    lax.fori_loop(0, PCH_PER_W, chunk_body, 0)


_pair_sc = pl.kernel(
    _pair_body,
    out_type=(jax.ShapeDtypeStruct((ETN, HD), jnp.float32),
              jax.ShapeDtypeStruct((ETN, HD), jnp.float32)),
    mesh=_SC_MESH,
    scratch_types=[
        pltpu.VMEM((CP,), jnp.int32),
        pltpu.VMEM((CP,), jnp.int32),
        pltpu.VMEM((CP, HD), jnp.float32),
        pltpu.VMEM((CP, HD), jnp.float32),
        pltpu.SemaphoreType.DMA,
        pltpu.SemaphoreType.DMA,
    ],
)


def _mm_body(x_ref, w_ref, o_ref):
    o_ref[...] = lax.dot_general(
        x_ref[...], w_ref[...], (((1,), (1,)), ((), ())),
        preferred_element_type=jnp.float32)


def _matmul_t(x, w, block_m=2000):
    M, K = x.shape
    H = w.shape[0]
    return pl.pallas_call(
        _mm_body,
        grid=(M // block_m,),
        in_specs=[pl.BlockSpec((block_m, K), lambda i: (i, 0)),
                  pl.BlockSpec((H, K), lambda i: (0, 0))],
        out_specs=pl.BlockSpec((block_m, H), lambda i: (i, 0)),
        out_shape=jax.ShapeDtypeStruct((M, H), jnp.float32),
    )(x, w)


def _post_body(agg_ref, s_ref, cb_ref, g_ref, b_ref, h_ref):
    t = jnp.maximum(agg_ref[...] + s_ref[...] + cb_ref[...], 0.0)
    mu = jnp.mean(t, axis=-1, keepdims=True)
    var = jnp.mean((t - mu) ** 2, axis=-1, keepdims=True)
    h_ref[...] = (t - mu) * lax.rsqrt(var + 1e-5) * g_ref[...] + b_ref[...]


def _post_conv(agg, s, conv_b, ln_g, ln_b, block_m=2000):
    M, H = agg.shape
    vec = pl.BlockSpec((1, H), lambda i: (0, 0))
    return pl.pallas_call(
        _post_body,
        grid=(M // block_m,),
        in_specs=[pl.BlockSpec((block_m, H), lambda i: (i, 0)),
                  pl.BlockSpec((block_m, H), lambda i: (i, 0)),
                  vec, vec, vec],
        out_specs=pl.BlockSpec((block_m, H), lambda i: (i, 0)),
        out_shape=jax.ShapeDtypeStruct((M, H), jnp.float32),
    )(agg, s, conv_b.reshape(1, H), ln_g.reshape(1, H), ln_b.reshape(1, H))


def _jk_body(h1_ref, h2_ref, agg3_ref, s3_ref, cb3_ref,
             w1_ref, w2_ref, w3_ref, bjk_ref, o_ref):
    h3 = agg3_ref[...] + s3_ref[...] + cb3_ref[...]
    acc = lax.dot_general(h1_ref[...], w1_ref[...], (((1,), (1,)), ((), ())),
                          preferred_element_type=jnp.float32)
    acc += lax.dot_general(h2_ref[...], w2_ref[...], (((1,), (1,)), ((), ())),
                           preferred_element_type=jnp.float32)
    acc += lax.dot_general(h3, w3_ref[...], (((1,), (1,)), ((), ())),
                           preferred_element_type=jnp.float32)
    o_ref[...] = acc + bjk_ref[...]


def _jk(h1, h2, agg3, s3, cb3, wjk, bjk, block_m=2000):
    M, H = h1.shape
    w1 = wjk[:, :H]
    w2 = wjk[:, H:2 * H]
    w3 = wjk[:, 2 * H:]
    blk = pl.BlockSpec((block_m, H), lambda i: (i, 0))
    wblk = pl.BlockSpec((H, H), lambda i: (0, 0))
    vec = pl.BlockSpec((1, H), lambda i: (0, 0))
    return pl.pallas_call(
        _jk_body,
        grid=(M // block_m,),
        in_specs=[blk, blk, blk, blk, vec, wblk, wblk, wblk, vec],
        out_specs=blk,
        out_shape=jax.ShapeDtypeStruct((M, H), jnp.float32),
    )(h1, h2, agg3, s3, cb3.reshape(1, H), w1, w2, w3, bjk.reshape(1, H))


def _mlp_body(pa_ref, pb_ref, po_ref, wpo_ref, b1_ref, w2_ref, b2_ref,
              w3_ref, b3_ref, o_ref):
    z1 = pa_ref[...] + pb_ref[...] + b1_ref[...]
    z1 += lax.dot_general(po_ref[...], wpo_ref[...], (((1,), (1,)), ((), ())),
                          preferred_element_type=jnp.float32)
    z1 = jnp.maximum(z1, 0.0)
    z2 = lax.dot_general(z1, w2_ref[...], (((1,), (1,)), ((), ())),
                         preferred_element_type=jnp.float32) + b2_ref[...]
    z2 = jnp.maximum(z2, 0.0)
    z = jnp.sum(z2 * w3_ref[...], axis=-1, keepdims=True) + b3_ref[...]
    o_ref[...] = jax.nn.sigmoid(z)


def _mlp(pa, pb, po, wpo, b1, w2, b2, w3, b3, block_m=2000):
    M, H = pa.shape
    H2 = w2.shape[0]
    return pl.pallas_call(
        _mlp_body,
        grid=(M // block_m,),
        in_specs=[pl.BlockSpec((block_m, H), lambda i: (i, 0)),
                  pl.BlockSpec((block_m, H), lambda i: (i, 0)),
                  pl.BlockSpec((block_m, 4), lambda i: (i, 0)),
                  pl.BlockSpec((H, 4), lambda i: (0, 0)),
                  pl.BlockSpec((1, H), lambda i: (0, 0)),
                  pl.BlockSpec((H2, H), lambda i: (0, 0)),
                  pl.BlockSpec((1, H2), lambda i: (0, 0)),
                  pl.BlockSpec((1, H2), lambda i: (0, 0)),
                  pl.BlockSpec((1, 1), lambda i: (0, 0))],
        out_specs=pl.BlockSpec((block_m, 1), lambda i: (i, 0)),
        out_shape=jax.ShapeDtypeStruct((M, 1), jnp.float32),
    )(pa, pb, po, wpo, b1, w2, b2, w3, b3)


def kernel(target_edge_index, x, embed_edge_index, pitch_score, onset_score,
           params):
    src, dst = embed_edge_index[0], embed_edge_index[1]
    convs = params['convs']
    zeros_half = jnp.zeros((NP, HH), jnp.float32)

    h = x
    hs = []
    agg3 = None
    s3 = None
    for i in range(3):
        p = convs[i]
        wall = jnp.concatenate([p['Wk'], p['Wq'], p['Wv'], p['Ws']], axis=0)
        kqvs = _matmul_t(h, wall)
        k = kqvs[:, :HD]
        q = kqvs[:, HD:2 * HD]
        v = kqvs[:, 2 * HD:3 * HD]
        s = kqvs[:, 3 * HD:]
        kt = jnp.concatenate([k[:, :HH], k[:, HH:]], axis=0)
        qvt = jnp.concatenate(
            [jnp.concatenate([q[:, :HH], v[:, :HH]], axis=1),
             jnp.concatenate([q[:, HH:], v[:, HH:]], axis=1)],
            axis=0)
        aggf = _edge_sc(kt, qvt, src, dst, zeros_half)
        agg = (aggf.reshape(NCORE, NP, HH)[:, :NN]
               .transpose(1, 0, 2).reshape(NN, HD))
        if i != 2:
            h = _post_conv(agg, s, p['b'], params['ln_g'], params['ln_b'])
            hs.append(h)
        else:
            agg3, s3 = agg, s

    hjk = _jk(hs[0], hs[1], agg3, s3, convs[2]['b'],
              params['Wjk'], params['bjk'])

    wa = params['Wp1'][:, :HD]
    wb = params['Wp1'][:, HD:2 * HD]
    wab = jnp.concatenate([wa, wb], axis=0)
    ab = _matmul_t(hjk, wab)
    tab = jnp.concatenate([ab[:, :HD], ab[:, HD:]], axis=0)

    ts, td = target_edge_index[0], target_edge_index[1]
    pa, pb = _pair_sc(tab, ts, td)

    po = jnp.concatenate(
        [pitch_score, onset_score,
         jnp.zeros((ETN, 1), jnp.float32)], axis=1)
    wpo = jnp.concatenate(
        [params['Wp1'][:, 2 * HD:],
         jnp.zeros((HD, 1), jnp.float32)], axis=1)

    return _mlp(pa, pb, po, wpo,
                params['bp1'].reshape(1, HD),
                params['Wp2'],
                params['bp2'].reshape(1, HD // 2),
                params['Wp3'].reshape(1, HD // 2),
                params['bp3'].reshape(1, 1))


# speedup vs baseline: 4.5277x; 1.1739x over previous
import functools

import jax
import jax.numpy as jnp
from jax import lax
from jax.experimental import pallas as pl
from jax.experimental.pallas import tpu as pltpu
from jax.experimental.pallas import tpu_sc as plsc

NN = 10000
EE = 320000
ETN = 100000
HD = 256
HH = 128
CE = 64
CP = 80
NSUB = 16
NCORE = 2
NP = 10048
ROWS_PER_SUB = 632
ECHUNKS = EE // CE
ECH_BASE = ECHUNKS // NSUB
ECH_REM = ECHUNKS % NSUB
PCHUNKS = ETN // CP
PCH_PER_W = -(-PCHUNKS // (NCORE * NSUB))

_SC_MESH = plsc.VectorSubcoreMesh(core_axis_name="c", subcore_axis_name="s")


NPAIR = ECH_BASE // 2


def _edge_body(kt, qvt, srch, dsth, zh, out,
               srcv0, srcv1, dstv0, dstv1, srcb0, srcb1, dstb0, dstb1,
               dsts0, dsts1, qv0, qv1, kb0, kb1, aggsh,
               semidx0, semidx1, semqv0, semqv1, semk0, semk1,
               semsc0, semsc1):
    c = lax.axis_index("c")
    s = lax.axis_index("s")
    rows0 = jnp.minimum(s * ROWS_PER_SUB, NP - ROWS_PER_SUB)
    bias = c * NN

    slots = (
        (srcv0, dstv0, srcb0, dstb0, dsts0, qv0, kb0, kb0,
         semidx0, semqv0, semk0, semsc0),
        (srcv1, dstv1, srcb1, dstb1, dsts1, qv1, kb1, kb1,
         semidx1, semqv1, semk1, semsc1),
    )

    def issue_idx(p, chunk):
        srcv, dstv = slots[p][0], slots[p][1]
        semidx = slots[p][8]
        base = (chunk * NSUB + s) * CE
        pltpu.async_copy(srch.at[pl.ds(base, CE)], srcv, semidx)
        pltpu.async_copy(dsth.at[pl.ds(base, CE)], dstv, semidx)

    def wait_idx(p):
        srcv, dstv = slots[p][0], slots[p][1]
        semidx = slots[p][8]
        pltpu.make_async_copy(srch.at[pl.ds(0, CE)], srcv, semidx).wait()
        pltpu.make_async_copy(dsth.at[pl.ds(0, CE)], dstv, semidx).wait()

    def bias_and_gather(p):
        srcv, dstv, srcb, dstb, dsts, qv, kb = slots[p][:7]
        semqv, semk = slots[p][9], slots[p][10]
        for jj in range(CE // 16):
            sl = pl.ds(jj * 16, 16)
            dv = dstv[sl]
            srcb[sl] = srcv[sl] + bias
            dstb[sl] = dv + bias
            dsts[sl] = dv
        pltpu.async_copy(qvt.at[srcb], qv, semqv)
        pltpu.async_copy(kt.at[dstb], kb, semk)

    def wait_gathers(p):
        srcb, dstb = slots[p][2], slots[p][3]
        qv, kb = slots[p][5], slots[p][6]
        semqv, semk = slots[p][9], slots[p][10]
        pltpu.make_async_copy(qvt.at[srcb], qv, semqv).wait()
        pltpu.make_async_copy(kt.at[dstb], kb, semk).wait()

    def compute(p):
        qv, kb, msg = slots[p][5], slots[p][6], slots[p][7]

        @plsc.parallel_loop(0, CE)
        def _(e):
            for hc in range(HH // 16):
                sl = pl.ds(hc * 16, 16)
                kvec = kb[e, sl]
                qvec = qv[e, sl]
                vvec = qv[e, pl.ds(HH + hc * 16, 16)]
                msg[e, sl] = vvec / (1.0 + jnp.exp(-(kvec + qvec)))

    def issue_scatter(p):
        dsts, msg, semsc = slots[p][4], slots[p][7], slots[p][11]
        pltpu.async_copy(msg, aggsh.at[dsts], semsc, add=True)

    def wait_scatter(p):
        dsts, msg, semsc = slots[p][4], slots[p][7], slots[p][11]
        pltpu.make_async_copy(msg, aggsh.at[dsts], semsc).wait()

    pltpu.sync_copy(zh.at[pl.ds(rows0, ROWS_PER_SUB)],
                    aggsh.at[pl.ds(rows0, ROWS_PER_SUB)])
    plsc.subcore_barrier()

    issue_idx(0, 0)
    issue_idx(1, 1)

    def pair_body(j, carry):
        wait_idx(0)

        @pl.when(j > 0)
        def _():
            wait_scatter(0)

        bias_and_gather(0)
        wait_idx(1)

        @pl.when(j > 0)
        def _():
            wait_scatter(1)

        bias_and_gather(1)

        @pl.when((j < NPAIR - 1) | (s < ECH_REM))
        def _():
            issue_idx(0, 2 * j + 2)

        @pl.when(j < NPAIR - 1)
        def _():
            issue_idx(1, 2 * j + 3)

        wait_gathers(0)
        compute(0)
        issue_scatter(0)
        wait_gathers(1)
        compute(1)
        issue_scatter(1)
        return carry

    lax.fori_loop(0, NPAIR, pair_body, 0)

    @pl.when(s < ECH_REM)
    def _():
        wait_idx(0)
        wait_scatter(0)
        bias_and_gather(0)
        wait_gathers(0)
        compute(0)
        issue_scatter(0)

    wait_scatter(0)
    wait_scatter(1)
    plsc.subcore_barrier()
    pltpu.sync_copy(aggsh.at[pl.ds(rows0, ROWS_PER_SUB)],
                    out.at[pl.ds(c * NP + rows0, ROWS_PER_SUB)])


_edge_sc = pl.kernel(
    _edge_body,
    out_type=jax.ShapeDtypeStruct((NCORE * NP, HH), jnp.float32),
    mesh=_SC_MESH,
    scratch_types=[
        pltpu.VMEM((CE,), jnp.int32),
        pltpu.VMEM((CE,), jnp.int32),
        pltpu.VMEM((CE,), jnp.int32),
        pltpu.VMEM((CE,), jnp.int32),
        pltpu.VMEM((CE,), jnp.int32),
        pltpu.VMEM((CE,), jnp.int32),
        pltpu.VMEM((CE,), jnp.int32),
        pltpu.VMEM((CE,), jnp.int32),
        pltpu.VMEM((CE,), jnp.int32),
        pltpu.VMEM((CE,), jnp.int32),
        pltpu.VMEM((CE, HD), jnp.float32),
        pltpu.VMEM((CE, HD), jnp.float32),
        pltpu.VMEM((CE, HH), jnp.float32),
        pltpu.VMEM((CE, HH), jnp.float32),
        pltpu.VMEM_SHARED((NP, HH), jnp.float32),
        pltpu.SemaphoreType.DMA,
        pltpu.SemaphoreType.DMA,
        pltpu.SemaphoreType.DMA,
        pltpu.SemaphoreType.DMA,
        pltpu.SemaphoreType.DMA,
        pltpu.SemaphoreType.DMA,
        pltpu.SemaphoreType.DMA,
        pltpu.SemaphoreType.DMA,
    ],
)


def _pair_body(tab, tsh, tdh, outa, outb, idx1, idx2, buf1, buf2, sem1, sem2):
    c = lax.axis_index("c")
    s = lax.axis_index("s")
    w = s * NCORE + c

    def chunk_body(j, carry):
        chunk = j * (NCORE * NSUB) + w

        @pl.when(chunk < PCHUNKS)
        def _():
            base = chunk * CP
            pltpu.sync_copy(tsh.at[pl.ds(base, CP)], idx1)
            pltpu.sync_copy(tdh.at[pl.ds(base, CP)], idx2)
            for jj in range(CP // 16):
                sl = pl.ds(jj * 16, 16)
                idx2[sl] = idx2[sl] + NN
            cp1 = pltpu.async_copy(tab.at[idx1], buf1, sem1)
            cp2 = pltpu.async_copy(tab.at[idx2], buf2, sem2)
            cp1.wait()
            cp2.wait()
            pltpu.sync_copy(buf1, outa.at[pl.ds(base, CP)])
            pltpu.sync_copy(buf2, outb.at[pl.ds(base, CP)])

        return carry

    lax.fori_loop(0, PCH_PER_W, chunk_body, 0)


_pair_sc = pl.kernel(
    _pair_body,
    out_type=(jax.ShapeDtypeStruct((ETN, HD), jnp.float32),
              jax.ShapeDtypeStruct((ETN, HD), jnp.float32)),
    mesh=_SC_MESH,
    scratch_types=[
        pltpu.VMEM((CP,), jnp.int32),
        pltpu.VMEM((CP,), jnp.int32),
        pltpu.VMEM((CP, HD), jnp.float32),
        pltpu.VMEM((CP, HD), jnp.float32),
        pltpu.SemaphoreType.DMA,
        pltpu.SemaphoreType.DMA,
    ],
)


def _mm_body(x_ref, w_ref, o_ref):
    o_ref[...] = lax.dot_general(
        x_ref[...], w_ref[...], (((1,), (1,)), ((), ())),
        preferred_element_type=jnp.float32)


def _matmul_t(x, w, block_m=2000):
    M, K = x.shape
    H = w.shape[0]
    return pl.pallas_call(
        _mm_body,
        grid=(M // block_m,),
        in_specs=[pl.BlockSpec((block_m, K), lambda i: (i, 0)),
                  pl.BlockSpec((H, K), lambda i: (0, 0))],
        out_specs=pl.BlockSpec((block_m, H), lambda i: (i, 0)),
        out_shape=jax.ShapeDtypeStruct((M, H), jnp.float32),
    )(x, w)


def _post_body(agg_ref, s_ref, cb_ref, g_ref, b_ref, h_ref):
    t = jnp.maximum(agg_ref[...] + s_ref[...] + cb_ref[...], 0.0)
    mu = jnp.mean(t, axis=-1, keepdims=True)
    var = jnp.mean((t - mu) ** 2, axis=-1, keepdims=True)
    h_ref[...] = (t - mu) * lax.rsqrt(var + 1e-5) * g_ref[...] + b_ref[...]


def _post_conv(agg, s, conv_b, ln_g, ln_b, block_m=2000):
    M, H = agg.shape
    vec = pl.BlockSpec((1, H), lambda i: (0, 0))
    return pl.pallas_call(
        _post_body,
        grid=(M // block_m,),
        in_specs=[pl.BlockSpec((block_m, H), lambda i: (i, 0)),
                  pl.BlockSpec((block_m, H), lambda i: (i, 0)),
                  vec, vec, vec],
        out_specs=pl.BlockSpec((block_m, H), lambda i: (i, 0)),
        out_shape=jax.ShapeDtypeStruct((M, H), jnp.float32),
    )(agg, s, conv_b.reshape(1, H), ln_g.reshape(1, H), ln_b.reshape(1, H))


def _jk_body(h1_ref, h2_ref, agg3_ref, s3_ref, cb3_ref,
             w1_ref, w2_ref, w3_ref, bjk_ref, o_ref):
    h3 = agg3_ref[...] + s3_ref[...] + cb3_ref[...]
    acc = lax.dot_general(h1_ref[...], w1_ref[...], (((1,), (1,)), ((), ())),
                          preferred_element_type=jnp.float32)
    acc += lax.dot_general(h2_ref[...], w2_ref[...], (((1,), (1,)), ((), ())),
                           preferred_element_type=jnp.float32)
    acc += lax.dot_general(h3, w3_ref[...], (((1,), (1,)), ((), ())),
                           preferred_element_type=jnp.float32)
    o_ref[...] = acc + bjk_ref[...]


def _jk(h1, h2, agg3, s3, cb3, wjk, bjk, block_m=2000):
    M, H = h1.shape
    w1 = wjk[:, :H]
    w2 = wjk[:, H:2 * H]
    w3 = wjk[:, 2 * H:]
    blk = pl.BlockSpec((block_m, H), lambda i: (i, 0))
    wblk = pl.BlockSpec((H, H), lambda i: (0, 0))
    vec = pl.BlockSpec((1, H), lambda i: (0, 0))
    return pl.pallas_call(
        _jk_body,
        grid=(M // block_m,),
        in_specs=[blk, blk, blk, blk, vec, wblk, wblk, wblk, vec],
        out_specs=blk,
        out_shape=jax.ShapeDtypeStruct((M, H), jnp.float32),
    )(h1, h2, agg3, s3, cb3.reshape(1, H), w1, w2, w3, bjk.reshape(1, H))


def _mlp_body(pa_ref, pb_ref, po_ref, wpo_ref, b1_ref, w2_ref, b2_ref,
              w3_ref, b3_ref, o_ref):
    z1 = pa_ref[...] + pb_ref[...] + b1_ref[...]
    z1 += lax.dot_general(po_ref[...], wpo_ref[...], (((1,), (1,)), ((), ())),
                          preferred_element_type=jnp.float32)
    z1 = jnp.maximum(z1, 0.0)
    z2 = lax.dot_general(z1, w2_ref[...], (((1,), (1,)), ((), ())),
                         preferred_element_type=jnp.float32) + b2_ref[...]
    z2 = jnp.maximum(z2, 0.0)
    z = jnp.sum(z2 * w3_ref[...], axis=-1, keepdims=True) + b3_ref[...]
    o_ref[...] = jax.nn.sigmoid(z)


def _mlp(pa, pb, po, wpo, b1, w2, b2, w3, b3, block_m=2000):
    M, H = pa.shape
    H2 = w2.shape[0]
    return pl.pallas_call(
        _mlp_body,
        grid=(M // block_m,),
        in_specs=[pl.BlockSpec((block_m, H), lambda i: (i, 0)),
                  pl.BlockSpec((block_m, H), lambda i: (i, 0)),
                  pl.BlockSpec((block_m, 4), lambda i: (i, 0)),
                  pl.BlockSpec((H, 4), lambda i: (0, 0)),
                  pl.BlockSpec((1, H), lambda i: (0, 0)),
                  pl.BlockSpec((H2, H), lambda i: (0, 0)),
                  pl.BlockSpec((1, H2), lambda i: (0, 0)),
                  pl.BlockSpec((1, H2), lambda i: (0, 0)),
                  pl.BlockSpec((1, 1), lambda i: (0, 0))],
        out_specs=pl.BlockSpec((block_m, 1), lambda i: (i, 0)),
        out_shape=jax.ShapeDtypeStruct((M, 1), jnp.float32),
    )(pa, pb, po, wpo, b1, w2, b2, w3, b3)


def kernel(target_edge_index, x, embed_edge_index, pitch_score, onset_score,
           params):
    src, dst = embed_edge_index[0], embed_edge_index[1]
    convs = params['convs']
    zeros_half = jnp.zeros((NP, HH), jnp.float32)

    h = x
    hs = []
    agg3 = None
    s3 = None
    for i in range(3):
        p = convs[i]
        wall = jnp.concatenate([p['Wk'], p['Wq'], p['Wv'], p['Ws']], axis=0)
        kqvs = _matmul_t(h, wall)
        k = kqvs[:, :HD]
        q = kqvs[:, HD:2 * HD]
        v = kqvs[:, 2 * HD:3 * HD]
        s = kqvs[:, 3 * HD:]
        kt = jnp.concatenate([k[:, :HH], k[:, HH:]], axis=0)
        qvt = jnp.concatenate(
            [jnp.concatenate([q[:, :HH], v[:, :HH]], axis=1),
             jnp.concatenate([q[:, HH:], v[:, HH:]], axis=1)],
            axis=0)
        aggf = _edge_sc(kt, qvt, src, dst, zeros_half)
        agg = (aggf.reshape(NCORE, NP, HH)[:, :NN]
               .transpose(1, 0, 2).reshape(NN, HD))
        if i != 2:
            h = _post_conv(agg, s, p['b'], params['ln_g'], params['ln_b'])
            hs.append(h)
        else:
            agg3, s3 = agg, s

    hjk = _jk(hs[0], hs[1], agg3, s3, convs[2]['b'],
              params['Wjk'], params['bjk'])

    wa = params['Wp1'][:, :HD]
    wb = params['Wp1'][:, HD:2 * HD]
    wab = jnp.concatenate([wa, wb], axis=0)
    ab = _matmul_t(hjk, wab)
    tab = jnp.concatenate([ab[:, :HD], ab[:, HD:]], axis=0)

    ts, td = target_edge_index[0], target_edge_index[1]
    pa, pb = _pair_sc(tab, ts, td)

    po = jnp.concatenate(
        [pitch_score, onset_score,
         jnp.zeros((ETN, 1), jnp.float32)], axis=1)
    wpo = jnp.concatenate(
        [params['Wp1'][:, 2 * HD:],
         jnp.zeros((HD, 1), jnp.float32)], axis=1)

    return _mlp(pa, pb, po, wpo,
                params['bp1'].reshape(1, HD),
                params['Wp2'],
                params['bp2'].reshape(1, HD // 2),
                params['Wp3'].reshape(1, HD // 2),
                params['bp3'].reshape(1, 1))


# speedup vs baseline: 4.5990x; 1.0158x over previous
import functools

import jax
import jax.numpy as jnp
from jax import lax
from jax.experimental import pallas as pl
from jax.experimental.pallas import tpu as pltpu
from jax.experimental.pallas import tpu_sc as plsc

NN = 10000
EE = 320000
ETN = 100000
HD = 256
HH = 128
CE = 64
CP = 80
NSUB = 16
NCORE = 2
NP = 10048
ROWS_PER_SUB = 632
ECHUNKS = EE // CE
ECH_BASE = ECHUNKS // NSUB
ECH_REM = ECHUNKS % NSUB
PCHUNKS = ETN // CP
PCH_PER_W = -(-PCHUNKS // (NCORE * NSUB))

_SC_MESH = plsc.VectorSubcoreMesh(core_axis_name="c", subcore_axis_name="s")


NPAIR = ECH_BASE // 2


def _edge_body(kt, qvt, srch, dsth, zh, out,
               srcv0, srcv1, dstv0, dstv1, srcb0, srcb1, dstb0, dstb1,
               dsts0, dsts1, qv0, qv1, kb0, kb1, aggsh,
               semidx0, semidx1, semqv0, semqv1, semk0, semk1,
               semsc0, semsc1):
    c = lax.axis_index("c")
    s = lax.axis_index("s")
    rows0 = jnp.minimum(s * ROWS_PER_SUB, NP - ROWS_PER_SUB)
    bias = c * NN

    slots = (
        (srcv0, dstv0, srcb0, dstb0, dsts0, qv0, kb0, kb0,
         semidx0, semqv0, semk0, semsc0),
        (srcv1, dstv1, srcb1, dstb1, dsts1, qv1, kb1, kb1,
         semidx1, semqv1, semk1, semsc1),
    )

    def issue_idx(p, chunk):
        srcv, dstv = slots[p][0], slots[p][1]
        semidx = slots[p][8]
        base = (chunk * NSUB + s) * CE
        pltpu.async_copy(srch.at[pl.ds(base, CE)], srcv, semidx)
        pltpu.async_copy(dsth.at[pl.ds(base, CE)], dstv, semidx)

    def wait_idx(p):
        srcv, dstv = slots[p][0], slots[p][1]
        semidx = slots[p][8]
        pltpu.make_async_copy(srch.at[pl.ds(0, CE)], srcv, semidx).wait()
        pltpu.make_async_copy(dsth.at[pl.ds(0, CE)], dstv, semidx).wait()

    def bias_and_gather(p):
        srcv, dstv, srcb, dstb, dsts, qv, kb = slots[p][:7]
        semqv, semk = slots[p][9], slots[p][10]
        for jj in range(CE // 16):
            sl = pl.ds(jj * 16, 16)
            dv = dstv[sl]
            srcb[sl] = srcv[sl] + bias
            dstb[sl] = dv + bias
            dsts[sl] = dv
        pltpu.async_copy(qvt.at[srcb], qv, semqv)
        pltpu.async_copy(kt.at[dstb], kb, semk)

    def wait_gathers(p):
        srcb, dstb = slots[p][2], slots[p][3]
        qv, kb = slots[p][5], slots[p][6]
        semqv, semk = slots[p][9], slots[p][10]
        pltpu.make_async_copy(qvt.at[srcb], qv, semqv).wait()
        pltpu.make_async_copy(kt.at[dstb], kb, semk).wait()

    def compute(p):
        qv, kb, msg = slots[p][5], slots[p][6], slots[p][7]

        @plsc.parallel_loop(0, CE)
        def _(e):
            for hc in range(HH // 16):
                sl = pl.ds(hc * 16, 16)
                kvec = kb[e, sl]
                qvec = qv[e, sl]
                vvec = qv[e, pl.ds(HH + hc * 16, 16)]
                msg[e, sl] = vvec / (1.0 + jnp.exp(-(kvec + qvec)))

    def issue_scatter(p):
        dsts, msg, semsc = slots[p][4], slots[p][7], slots[p][11]
        pltpu.async_copy(msg, aggsh.at[dsts], semsc, add=True)

    def wait_scatter(p):
        dsts, msg, semsc = slots[p][4], slots[p][7], slots[p][11]
        pltpu.make_async_copy(msg, aggsh.at[dsts], semsc).wait()

    pltpu.sync_copy(zh.at[pl.ds(rows0, ROWS_PER_SUB)],
                    aggsh.at[pl.ds(rows0, ROWS_PER_SUB)])
    plsc.subcore_barrier()

    issue_idx(0, 0)
    issue_idx(1, 1)

    def pair_body(j, carry):
        wait_idx(0)

        @pl.when(j > 0)
        def _():
            wait_scatter(0)

        bias_and_gather(0)
        wait_idx(1)

        @pl.when(j > 0)
        def _():
            wait_scatter(1)

        bias_and_gather(1)

        @pl.when((j < NPAIR - 1) | (s < ECH_REM))
        def _():
            issue_idx(0, 2 * j + 2)

        @pl.when(j < NPAIR - 1)
        def _():
            issue_idx(1, 2 * j + 3)

        wait_gathers(0)
        compute(0)
        issue_scatter(0)
        wait_gathers(1)
        compute(1)
        issue_scatter(1)
        return carry

    lax.fori_loop(0, NPAIR, pair_body, 0)

    @pl.when(s < ECH_REM)
    def _():
        wait_idx(0)
        wait_scatter(0)
        bias_and_gather(0)
        wait_gathers(0)
        compute(0)
        issue_scatter(0)

    wait_scatter(0)
    wait_scatter(1)
    plsc.subcore_barrier()
    pltpu.sync_copy(aggsh.at[pl.ds(rows0, ROWS_PER_SUB)],
                    out.at[pl.ds(c * NP + rows0, ROWS_PER_SUB)])


_edge_sc = pl.kernel(
    _edge_body,
    out_type=jax.ShapeDtypeStruct((NCORE * NP, HH), jnp.float32),
    mesh=_SC_MESH,
    scratch_types=[
        pltpu.VMEM((CE,), jnp.int32),
        pltpu.VMEM((CE,), jnp.int32),
        pltpu.VMEM((CE,), jnp.int32),
        pltpu.VMEM((CE,), jnp.int32),
        pltpu.VMEM((CE,), jnp.int32),
        pltpu.VMEM((CE,), jnp.int32),
        pltpu.VMEM((CE,), jnp.int32),
        pltpu.VMEM((CE,), jnp.int32),
        pltpu.VMEM((CE,), jnp.int32),
        pltpu.VMEM((CE,), jnp.int32),
        pltpu.VMEM((CE, HD), jnp.float32),
        pltpu.VMEM((CE, HD), jnp.float32),
        pltpu.VMEM((CE, HH), jnp.float32),
        pltpu.VMEM((CE, HH), jnp.float32),
        pltpu.VMEM_SHARED((NP, HH), jnp.float32),
        pltpu.SemaphoreType.DMA,
        pltpu.SemaphoreType.DMA,
        pltpu.SemaphoreType.DMA,
        pltpu.SemaphoreType.DMA,
        pltpu.SemaphoreType.DMA,
        pltpu.SemaphoreType.DMA,
        pltpu.SemaphoreType.DMA,
        pltpu.SemaphoreType.DMA,
    ],
)


NWORK = NCORE * NSUB
PPAIR = 19
PTAIL0 = 2 * PPAIR
PTAIL1 = PTAIL0 + 1


def _pair_body(tab, tsh, tdh, outa, outb,
               ts0, ts1, td0, td1, bufa0, bufa1, bufb0, bufb1,
               semi0, semi1, sema0, sema1, semb0, semb1):
    c = lax.axis_index("c")
    s = lax.axis_index("s")
    w = s * NCORE + c
    rem = PCHUNKS % NWORK

    slots = (
        (ts0, td0, bufa0, bufb0, semi0, sema0, semb0),
        (ts1, td1, bufa1, bufb1, semi1, sema1, semb1),
    )

    def issue_idx(p, i):
        tsv, tdv, semi = slots[p][0], slots[p][1], slots[p][4]
        base = (i * NWORK + w) * CP
        pltpu.async_copy(tsh.at[pl.ds(base, CP)], tsv, semi)
        pltpu.async_copy(tdh.at[pl.ds(base, CP)], tdv, semi)

    def wait_idx(p):
        tsv, tdv, semi = slots[p][0], slots[p][1], slots[p][4]
        pltpu.make_async_copy(tsh.at[pl.ds(0, CP)], tsv, semi).wait()
        pltpu.make_async_copy(tdh.at[pl.ds(0, CP)], tdv, semi).wait()

    def bias_and_gather(p):
        tsv, tdv, bufa, bufb, _, sema, semb = slots[p]
        for jj in range(CP // 16):
            sl = pl.ds(jj * 16, 16)
            tdv[sl] = tdv[sl] + NN
        pltpu.async_copy(tab.at[tsv], bufa, sema)
        pltpu.async_copy(tab.at[tdv], bufb, semb)

    def wait_and_store(p, i):
        tsv, tdv, bufa, bufb, _, sema, semb = slots[p]
        base = (i * NWORK + w) * CP
        pltpu.make_async_copy(tab.at[tsv], bufa, sema).wait()
        pltpu.make_async_copy(tab.at[tdv], bufb, semb).wait()
        pltpu.sync_copy(bufa, outa.at[pl.ds(base, CP)])
        pltpu.sync_copy(bufb, outb.at[pl.ds(base, CP)])

    issue_idx(0, 0)
    issue_idx(1, 1)

    def pair_body(j, carry):
        wait_idx(0)
        bias_and_gather(0)
        wait_idx(1)
        bias_and_gather(1)
        wait_and_store(0, 2 * j)
        issue_idx(0, 2 * j + 2)
        wait_and_store(1, 2 * j + 1)

        @pl.when((j < PPAIR - 1) | (w < rem))
        def _():
            issue_idx(1, 2 * j + 3)

        return carry

    lax.fori_loop(0, PPAIR, pair_body, 0)

    wait_idx(0)
    bias_and_gather(0)
    wait_and_store(0, PTAIL0)

    @pl.when(w < rem)
    def _():
        wait_idx(1)
        bias_and_gather(1)
        wait_and_store(1, PTAIL1)


_pair_sc = pl.kernel(
    _pair_body,
    out_type=(jax.ShapeDtypeStruct((ETN, HD), jnp.float32),
              jax.ShapeDtypeStruct((ETN, HD), jnp.float32)),
    mesh=_SC_MESH,
    scratch_types=[
        pltpu.VMEM((CP,), jnp.int32),
        pltpu.VMEM((CP,), jnp.int32),
        pltpu.VMEM((CP,), jnp.int32),
        pltpu.VMEM((CP,), jnp.int32),
        pltpu.VMEM((CP, HD), jnp.float32),
        pltpu.VMEM((CP, HD), jnp.float32),
        pltpu.VMEM((CP, HD), jnp.float32),
        pltpu.VMEM((CP, HD), jnp.float32),
        pltpu.SemaphoreType.DMA,
        pltpu.SemaphoreType.DMA,
        pltpu.SemaphoreType.DMA,
        pltpu.SemaphoreType.DMA,
        pltpu.SemaphoreType.DMA,
        pltpu.SemaphoreType.DMA,
    ],
)


def _mm_body(x_ref, w_ref, o_ref):
    o_ref[...] = lax.dot_general(
        x_ref[...], w_ref[...], (((1,), (1,)), ((), ())),
        preferred_element_type=jnp.float32)


def _matmul_t(x, w, block_m=2000):
    M, K = x.shape
    H = w.shape[0]
    return pl.pallas_call(
        _mm_body,
        grid=(M // block_m,),
        in_specs=[pl.BlockSpec((block_m, K), lambda i: (i, 0)),
                  pl.BlockSpec((H, K), lambda i: (0, 0))],
        out_specs=pl.BlockSpec((block_m, H), lambda i: (i, 0)),
        out_shape=jax.ShapeDtypeStruct((M, H), jnp.float32),
    )(x, w)


def _post_body(agg_ref, s_ref, cb_ref, g_ref, b_ref, h_ref):
    t = jnp.maximum(agg_ref[...] + s_ref[...] + cb_ref[...], 0.0)
    mu = jnp.mean(t, axis=-1, keepdims=True)
    var = jnp.mean((t - mu) ** 2, axis=-1, keepdims=True)
    h_ref[...] = (t - mu) * lax.rsqrt(var + 1e-5) * g_ref[...] + b_ref[...]


def _post_conv(agg, s, conv_b, ln_g, ln_b, block_m=2000):
    M, H = agg.shape
    vec = pl.BlockSpec((1, H), lambda i: (0, 0))
    return pl.pallas_call(
        _post_body,
        grid=(M // block_m,),
        in_specs=[pl.BlockSpec((block_m, H), lambda i: (i, 0)),
                  pl.BlockSpec((block_m, H), lambda i: (i, 0)),
                  vec, vec, vec],
        out_specs=pl.BlockSpec((block_m, H), lambda i: (i, 0)),
        out_shape=jax.ShapeDtypeStruct((M, H), jnp.float32),
    )(agg, s, conv_b.reshape(1, H), ln_g.reshape(1, H), ln_b.reshape(1, H))


def _jk_body(h1_ref, h2_ref, agg3_ref, s3_ref, cb3_ref,
             w1_ref, w2_ref, w3_ref, bjk_ref, o_ref):
    h3 = agg3_ref[...] + s3_ref[...] + cb3_ref[...]
    acc = lax.dot_general(h1_ref[...], w1_ref[...], (((1,), (1,)), ((), ())),
                          preferred_element_type=jnp.float32)
    acc += lax.dot_general(h2_ref[...], w2_ref[...], (((1,), (1,)), ((), ())),
                           preferred_element_type=jnp.float32)
    acc += lax.dot_general(h3, w3_ref[...], (((1,), (1,)), ((), ())),
                           preferred_element_type=jnp.float32)
    o_ref[...] = acc + bjk_ref[...]


def _jk(h1, h2, agg3, s3, cb3, wjk, bjk, block_m=2000):
    M, H = h1.shape
    w1 = wjk[:, :H]
    w2 = wjk[:, H:2 * H]
    w3 = wjk[:, 2 * H:]
    blk = pl.BlockSpec((block_m, H), lambda i: (i, 0))
    wblk = pl.BlockSpec((H, H), lambda i: (0, 0))
    vec = pl.BlockSpec((1, H), lambda i: (0, 0))
    return pl.pallas_call(
        _jk_body,
        grid=(M // block_m,),
        in_specs=[blk, blk, blk, blk, vec, wblk, wblk, wblk, vec],
        out_specs=blk,
        out_shape=jax.ShapeDtypeStruct((M, H), jnp.float32),
    )(h1, h2, agg3, s3, cb3.reshape(1, H), w1, w2, w3, bjk.reshape(1, H))


def _mlp_body(pa_ref, pb_ref, po_ref, wpo_ref, b1_ref, w2_ref, b2_ref,
              w3_ref, b3_ref, o_ref):
    z1 = pa_ref[...] + pb_ref[...] + b1_ref[...]
    z1 += lax.dot_general(po_ref[...], wpo_ref[...], (((1,), (1,)), ((), ())),
                          preferred_element_type=jnp.float32)
    z1 = jnp.maximum(z1, 0.0)
    z2 = lax.dot_general(z1, w2_ref[...], (((1,), (1,)), ((), ())),
                         preferred_element_type=jnp.float32) + b2_ref[...]
    z2 = jnp.maximum(z2, 0.0)
    z = jnp.sum(z2 * w3_ref[...], axis=-1, keepdims=True) + b3_ref[...]
    o_ref[...] = jax.nn.sigmoid(z)


def _mlp(pa, pb, po, wpo, b1, w2, b2, w3, b3, block_m=2000):
    M, H = pa.shape
    H2 = w2.shape[0]
    return pl.pallas_call(
        _mlp_body,
        grid=(M // block_m,),
        in_specs=[pl.BlockSpec((block_m, H), lambda i: (i, 0)),
                  pl.BlockSpec((block_m, H), lambda i: (i, 0)),
                  pl.BlockSpec((block_m, 4), lambda i: (i, 0)),
                  pl.BlockSpec((H, 4), lambda i: (0, 0)),
                  pl.BlockSpec((1, H), lambda i: (0, 0)),
                  pl.BlockSpec((H2, H), lambda i: (0, 0)),
                  pl.BlockSpec((1, H2), lambda i: (0, 0)),
                  pl.BlockSpec((1, H2), lambda i: (0, 0)),
                  pl.BlockSpec((1, 1), lambda i: (0, 0))],
        out_specs=pl.BlockSpec((block_m, 1), lambda i: (i, 0)),
        out_shape=jax.ShapeDtypeStruct((M, 1), jnp.float32),
    )(pa, pb, po, wpo, b1, w2, b2, w3, b3)


def kernel(target_edge_index, x, embed_edge_index, pitch_score, onset_score,
           params):
    src, dst = embed_edge_index[0], embed_edge_index[1]
    convs = params['convs']
    zeros_half = jnp.zeros((NP, HH), jnp.float32)

    h = x
    hs = []
    agg3 = None
    s3 = None
    for i in range(3):
        p = convs[i]
        wall = jnp.concatenate([p['Wk'], p['Wq'], p['Wv'], p['Ws']], axis=0)
        kqvs = _matmul_t(h, wall)
        k = kqvs[:, :HD]
        q = kqvs[:, HD:2 * HD]
        v = kqvs[:, 2 * HD:3 * HD]
        s = kqvs[:, 3 * HD:]
        kt = jnp.concatenate([k[:, :HH], k[:, HH:]], axis=0)
        qvt = jnp.concatenate(
            [jnp.concatenate([q[:, :HH], v[:, :HH]], axis=1),
             jnp.concatenate([q[:, HH:], v[:, HH:]], axis=1)],
            axis=0)
        aggf = _edge_sc(kt, qvt, src, dst, zeros_half)
        agg = (aggf.reshape(NCORE, NP, HH)[:, :NN]
               .transpose(1, 0, 2).reshape(NN, HD))
        if i != 2:
            h = _post_conv(agg, s, p['b'], params['ln_g'], params['ln_b'])
            hs.append(h)
        else:
            agg3, s3 = agg, s

    hjk = _jk(hs[0], hs[1], agg3, s3, convs[2]['b'],
              params['Wjk'], params['bjk'])

    wa = params['Wp1'][:, :HD]
    wb = params['Wp1'][:, HD:2 * HD]
    wab = jnp.concatenate([wa, wb], axis=0)
    ab = _matmul_t(hjk, wab)
    tab = jnp.concatenate([ab[:, :HD], ab[:, HD:]], axis=0)

    ts, td = target_edge_index[0], target_edge_index[1]
    pa, pb = _pair_sc(tab, ts, td)

    po = jnp.concatenate(
        [pitch_score, onset_score,
         jnp.zeros((ETN, 1), jnp.float32)], axis=1)
    wpo = jnp.concatenate(
        [params['Wp1'][:, 2 * HD:],
         jnp.zeros((HD, 1), jnp.float32)], axis=1)

    return _mlp(pa, pb, po, wpo,
                params['bp1'].reshape(1, HD),
                params['Wp2'],
                params['bp2'].reshape(1, HD // 2),
                params['Wp3'].reshape(1, HD // 2),
                params['bp3'].reshape(1, 1))


# speedup vs baseline: 4.5991x; 1.0000x over previous
import jax
import jax.numpy as jnp
from jax import lax
from jax.experimental import pallas as pl
from jax.experimental.pallas import tpu as pltpu
from jax.experimental.pallas import tpu_sc as plsc

NN = 10000
EE = 320000
ETN = 100000
HD = 256
HH = 128
CE = 64
CP = 80
NSUB = 16
NCORE = 2
NP = 10048
ROWS_PER_SUB = 632
ECHUNKS = EE // CE
ECH_BASE = ECHUNKS // NSUB
ECH_REM = ECHUNKS % NSUB
PCHUNKS = ETN // CP

_SC_MESH = plsc.VectorSubcoreMesh(core_axis_name="c", subcore_axis_name="s")


NPAIR = ECH_BASE // 2


def _edge_body(kt, qvt, srch, dsth, zh, out,
               srcv0, srcv1, dstv0, dstv1, srcb0, srcb1, dstb0, dstb1,
               dsts0, dsts1, qv0, qv1, kb0, kb1, aggsh,
               semidx0, semidx1, semqv0, semqv1, semk0, semk1,
               semsc0, semsc1):
    c = lax.axis_index("c")
    s = lax.axis_index("s")
    rows0 = jnp.minimum(s * ROWS_PER_SUB, NP - ROWS_PER_SUB)
    bias = c * NN

    slots = (
        (srcv0, dstv0, srcb0, dstb0, dsts0, qv0, kb0, kb0,
         semidx0, semqv0, semk0, semsc0),
        (srcv1, dstv1, srcb1, dstb1, dsts1, qv1, kb1, kb1,
         semidx1, semqv1, semk1, semsc1),
    )

    def issue_idx(p, chunk):
        srcv, dstv = slots[p][0], slots[p][1]
        semidx = slots[p][8]
        base = (chunk * NSUB + s) * CE
        pltpu.async_copy(srch.at[pl.ds(base, CE)], srcv, semidx)
        pltpu.async_copy(dsth.at[pl.ds(base, CE)], dstv, semidx)

    def wait_idx(p):
        srcv, dstv = slots[p][0], slots[p][1]
        semidx = slots[p][8]
        pltpu.make_async_copy(srch.at[pl.ds(0, CE)], srcv, semidx).wait()
        pltpu.make_async_copy(dsth.at[pl.ds(0, CE)], dstv, semidx).wait()

    def bias_and_gather(p):
        srcv, dstv, srcb, dstb, dsts, qv, kb = slots[p][:7]
        semqv, semk = slots[p][9], slots[p][10]
        for jj in range(CE // 16):
            sl = pl.ds(jj * 16, 16)
            dv = dstv[sl]
            srcb[sl] = srcv[sl] + bias
            dstb[sl] = dv + bias
            dsts[sl] = dv
        pltpu.async_copy(qvt.at[srcb], qv, semqv)
        pltpu.async_copy(kt.at[dstb], kb, semk)

    def wait_gathers(p):
        srcb, dstb = slots[p][2], slots[p][3]
        qv, kb = slots[p][5], slots[p][6]
        semqv, semk = slots[p][9], slots[p][10]
        pltpu.make_async_copy(qvt.at[srcb], qv, semqv).wait()
        pltpu.make_async_copy(kt.at[dstb], kb, semk).wait()

    def compute(p):
        qv, kb, msg = slots[p][5], slots[p][6], slots[p][7]

        @plsc.parallel_loop(0, CE)
        def _(e):
            for hc in range(HH // 16):
                sl = pl.ds(hc * 16, 16)
                kvec = kb[e, sl]
                qvec = qv[e, sl]
                vvec = qv[e, pl.ds(HH + hc * 16, 16)]
                msg[e, sl] = vvec / (1.0 + jnp.exp(-(kvec + qvec)))

    def issue_scatter(p):
        dsts, msg, semsc = slots[p][4], slots[p][7], slots[p][11]
        pltpu.async_copy(msg, aggsh.at[dsts], semsc, add=True)

    def wait_scatter(p):
        dsts, msg, semsc = slots[p][4], slots[p][7], slots[p][11]
        pltpu.make_async_copy(msg, aggsh.at[dsts], semsc).wait()

    pltpu.sync_copy(zh.at[pl.ds(rows0, ROWS_PER_SUB)],
                    aggsh.at[pl.ds(rows0, ROWS_PER_SUB)])
    plsc.subcore_barrier()

    issue_idx(0, 0)
    issue_idx(1, 1)

    def pair_body(j, carry):
        wait_idx(0)

        @pl.when(j > 0)
        def _():
            wait_scatter(0)

        bias_and_gather(0)
        wait_idx(1)

        @pl.when(j > 0)
        def _():
            wait_scatter(1)

        bias_and_gather(1)

        @pl.when((j < NPAIR - 1) | (s < ECH_REM))
        def _():
            issue_idx(0, 2 * j + 2)

        @pl.when(j < NPAIR - 1)
        def _():
            issue_idx(1, 2 * j + 3)

        wait_gathers(0)
        compute(0)
        issue_scatter(0)
        wait_gathers(1)
        compute(1)
        issue_scatter(1)
        return carry

    lax.fori_loop(0, NPAIR, pair_body, 0)

    @pl.when(s < ECH_REM)
    def _():
        wait_idx(0)
        wait_scatter(0)
        bias_and_gather(0)
        wait_gathers(0)
        compute(0)
        issue_scatter(0)

    wait_scatter(0)
    wait_scatter(1)
    plsc.subcore_barrier()
    pltpu.sync_copy(aggsh.at[pl.ds(rows0, ROWS_PER_SUB)],
                    out.at[pl.ds(c * NP + rows0, ROWS_PER_SUB)])


_edge_sc = pl.kernel(
    _edge_body,
    out_type=jax.ShapeDtypeStruct((NCORE * NP, HH), jnp.float32),
    mesh=_SC_MESH,
    scratch_types=[
        pltpu.VMEM((CE,), jnp.int32),
        pltpu.VMEM((CE,), jnp.int32),
        pltpu.VMEM((CE,), jnp.int32),
        pltpu.VMEM((CE,), jnp.int32),
        pltpu.VMEM((CE,), jnp.int32),
        pltpu.VMEM((CE,), jnp.int32),
        pltpu.VMEM((CE,), jnp.int32),
        pltpu.VMEM((CE,), jnp.int32),
        pltpu.VMEM((CE,), jnp.int32),
        pltpu.VMEM((CE,), jnp.int32),
        pltpu.VMEM((CE, HD), jnp.float32),
        pltpu.VMEM((CE, HD), jnp.float32),
        pltpu.VMEM((CE, HH), jnp.float32),
        pltpu.VMEM((CE, HH), jnp.float32),
        pltpu.VMEM_SHARED((NP, HH), jnp.float32),
        pltpu.SemaphoreType.DMA,
        pltpu.SemaphoreType.DMA,
        pltpu.SemaphoreType.DMA,
        pltpu.SemaphoreType.DMA,
        pltpu.SemaphoreType.DMA,
        pltpu.SemaphoreType.DMA,
        pltpu.SemaphoreType.DMA,
        pltpu.SemaphoreType.DMA,
    ],
)


NWORK = NCORE * NSUB
PPAIR = 19
PTAIL0 = 2 * PPAIR
PTAIL1 = PTAIL0 + 1


def _pair_body(tab, tsh, tdh, outa, outb,
               ts0, ts1, td0, td1, bufa0, bufa1, bufb0, bufb1,
               semi0, semi1, sema0, sema1, semb0, semb1):
    c = lax.axis_index("c")
    s = lax.axis_index("s")
    w = s * NCORE + c
    rem = PCHUNKS % NWORK

    slots = (
        (ts0, td0, bufa0, bufb0, semi0, sema0, semb0),
        (ts1, td1, bufa1, bufb1, semi1, sema1, semb1),
    )

    def issue_idx(p, i):
        tsv, tdv, semi = slots[p][0], slots[p][1], slots[p][4]
        base = (i * NWORK + w) * CP
        pltpu.async_copy(tsh.at[pl.ds(base, CP)], tsv, semi)
        pltpu.async_copy(tdh.at[pl.ds(base, CP)], tdv, semi)

    def wait_idx(p):
        tsv, tdv, semi = slots[p][0], slots[p][1], slots[p][4]
        pltpu.make_async_copy(tsh.at[pl.ds(0, CP)], tsv, semi).wait()
        pltpu.make_async_copy(tdh.at[pl.ds(0, CP)], tdv, semi).wait()

    def bias_and_gather(p):
        tsv, tdv, bufa, bufb, _, sema, semb = slots[p]
        for jj in range(CP // 16):
            sl = pl.ds(jj * 16, 16)
            tdv[sl] = tdv[sl] + NN
        pltpu.async_copy(tab.at[tsv], bufa, sema)
        pltpu.async_copy(tab.at[tdv], bufb, semb)

    def wait_and_store(p, i):
        tsv, tdv, bufa, bufb, _, sema, semb = slots[p]
        base = (i * NWORK + w) * CP
        pltpu.make_async_copy(tab.at[tsv], bufa, sema).wait()
        pltpu.make_async_copy(tab.at[tdv], bufb, semb).wait()
        pltpu.sync_copy(bufa, outa.at[pl.ds(base, CP)])
        pltpu.sync_copy(bufb, outb.at[pl.ds(base, CP)])

    issue_idx(0, 0)
    issue_idx(1, 1)

    def pair_body(j, carry):
        wait_idx(0)
        bias_and_gather(0)
        wait_idx(1)
        bias_and_gather(1)
        wait_and_store(0, 2 * j)
        issue_idx(0, 2 * j + 2)
        wait_and_store(1, 2 * j + 1)

        @pl.when((j < PPAIR - 1) | (w < rem))
        def _():
            issue_idx(1, 2 * j + 3)

        return carry

    lax.fori_loop(0, PPAIR, pair_body, 0)

    wait_idx(0)
    bias_and_gather(0)
    wait_and_store(0, PTAIL0)

    @pl.when(w < rem)
    def _():
        wait_idx(1)
        bias_and_gather(1)
        wait_and_store(1, PTAIL1)


_pair_sc = pl.kernel(
    _pair_body,
    out_type=(jax.ShapeDtypeStruct((ETN, HD), jnp.float32),
              jax.ShapeDtypeStruct((ETN, HD), jnp.float32)),
    mesh=_SC_MESH,
    scratch_types=[
        pltpu.VMEM((CP,), jnp.int32),
        pltpu.VMEM((CP,), jnp.int32),
        pltpu.VMEM((CP,), jnp.int32),
        pltpu.VMEM((CP,), jnp.int32),
        pltpu.VMEM((CP, HD), jnp.float32),
        pltpu.VMEM((CP, HD), jnp.float32),
        pltpu.VMEM((CP, HD), jnp.float32),
        pltpu.VMEM((CP, HD), jnp.float32),
        pltpu.SemaphoreType.DMA,
        pltpu.SemaphoreType.DMA,
        pltpu.SemaphoreType.DMA,
        pltpu.SemaphoreType.DMA,
        pltpu.SemaphoreType.DMA,
        pltpu.SemaphoreType.DMA,
    ],
)


def _mm_body(x_ref, w_ref, o_ref):
    o_ref[...] = lax.dot_general(
        x_ref[...], w_ref[...], (((1,), (1,)), ((), ())),
        preferred_element_type=jnp.float32)


def _matmul_t(x, w, block_m=2000):
    M, K = x.shape
    H = w.shape[0]
    return pl.pallas_call(
        _mm_body,
        grid=(M // block_m,),
        in_specs=[pl.BlockSpec((block_m, K), lambda i: (i, 0)),
                  pl.BlockSpec((H, K), lambda i: (0, 0))],
        out_specs=pl.BlockSpec((block_m, H), lambda i: (i, 0)),
        out_shape=jax.ShapeDtypeStruct((M, H), jnp.float32),
    )(x, w)


def _post_body(agg_ref, s_ref, cb_ref, g_ref, b_ref, h_ref):
    t = jnp.maximum(agg_ref[...] + s_ref[...] + cb_ref[...], 0.0)
    mu = jnp.mean(t, axis=-1, keepdims=True)
    var = jnp.mean((t - mu) ** 2, axis=-1, keepdims=True)
    h_ref[...] = (t - mu) * lax.rsqrt(var + 1e-5) * g_ref[...] + b_ref[...]


def _post_conv(agg, s, conv_b, ln_g, ln_b, block_m=2000):
    M, H = agg.shape
    vec = pl.BlockSpec((1, H), lambda i: (0, 0))
    return pl.pallas_call(
        _post_body,
        grid=(M // block_m,),
        in_specs=[pl.BlockSpec((block_m, H), lambda i: (i, 0)),
                  pl.BlockSpec((block_m, H), lambda i: (i, 0)),
                  vec, vec, vec],
        out_specs=pl.BlockSpec((block_m, H), lambda i: (i, 0)),
        out_shape=jax.ShapeDtypeStruct((M, H), jnp.float32),
    )(agg, s, conv_b.reshape(1, H), ln_g.reshape(1, H), ln_b.reshape(1, H))


def _jk_body(h1_ref, h2_ref, agg3_ref, s3_ref, cb3_ref,
             w1_ref, w2_ref, w3_ref, bjk_ref, o_ref):
    h3 = agg3_ref[...] + s3_ref[...] + cb3_ref[...]
    acc = lax.dot_general(h1_ref[...], w1_ref[...], (((1,), (1,)), ((), ())),
                          preferred_element_type=jnp.float32)
    acc += lax.dot_general(h2_ref[...], w2_ref[...], (((1,), (1,)), ((), ())),
                           preferred_element_type=jnp.float32)
    acc += lax.dot_general(h3, w3_ref[...], (((1,), (1,)), ((), ())),
                           preferred_element_type=jnp.float32)
    o_ref[...] = acc + bjk_ref[...]


def _jk(h1, h2, agg3, s3, cb3, wjk, bjk, block_m=2000):
    M, H = h1.shape
    w1 = wjk[:, :H]
    w2 = wjk[:, H:2 * H]
    w3 = wjk[:, 2 * H:]
    blk = pl.BlockSpec((block_m, H), lambda i: (i, 0))
    wblk = pl.BlockSpec((H, H), lambda i: (0, 0))
    vec = pl.BlockSpec((1, H), lambda i: (0, 0))
    return pl.pallas_call(
        _jk_body,
        grid=(M // block_m,),
        in_specs=[blk, blk, blk, blk, vec, wblk, wblk, wblk, vec],
        out_specs=blk,
        out_shape=jax.ShapeDtypeStruct((M, H), jnp.float32),
    )(h1, h2, agg3, s3, cb3.reshape(1, H), w1, w2, w3, bjk.reshape(1, H))


def _mlp_body(pa_ref, pb_ref, po_ref, wpo_ref, b1_ref, w2_ref, b2_ref,
              w3_ref, b3_ref, o_ref):
    z1 = pa_ref[...] + pb_ref[...] + b1_ref[...]
    z1 += lax.dot_general(po_ref[...], wpo_ref[...], (((1,), (1,)), ((), ())),
                          preferred_element_type=jnp.float32)
    z1 = jnp.maximum(z1, 0.0)
    z2 = lax.dot_general(z1, w2_ref[...], (((1,), (1,)), ((), ())),
                         preferred_element_type=jnp.float32) + b2_ref[...]
    z2 = jnp.maximum(z2, 0.0)
    z = jnp.sum(z2 * w3_ref[...], axis=-1, keepdims=True) + b3_ref[...]
    o_ref[...] = jax.nn.sigmoid(z)


def _mlp(pa, pb, po, wpo, b1, w2, b2, w3, b3, block_m=2000):
    M, H = pa.shape
    H2 = w2.shape[0]
    return pl.pallas_call(
        _mlp_body,
        grid=(M // block_m,),
        in_specs=[pl.BlockSpec((block_m, H), lambda i: (i, 0)),
                  pl.BlockSpec((block_m, H), lambda i: (i, 0)),
                  pl.BlockSpec((block_m, 4), lambda i: (i, 0)),
                  pl.BlockSpec((H, 4), lambda i: (0, 0)),
                  pl.BlockSpec((1, H), lambda i: (0, 0)),
                  pl.BlockSpec((H2, H), lambda i: (0, 0)),
                  pl.BlockSpec((1, H2), lambda i: (0, 0)),
                  pl.BlockSpec((1, H2), lambda i: (0, 0)),
                  pl.BlockSpec((1, 1), lambda i: (0, 0))],
        out_specs=pl.BlockSpec((block_m, 1), lambda i: (i, 0)),
        out_shape=jax.ShapeDtypeStruct((M, 1), jnp.float32),
    )(pa, pb, po, wpo, b1, w2, b2, w3, b3)


def kernel(target_edge_index, x, embed_edge_index, pitch_score, onset_score,
           params):
    src, dst = embed_edge_index[0], embed_edge_index[1]
    convs = params['convs']
    zeros_half = jnp.zeros((NP, HH), jnp.float32)

    h = x
    hs = []
    agg3 = None
    s3 = None
    for i in range(3):
        p = convs[i]
        wall = jnp.concatenate([p['Wk'], p['Wq'], p['Wv'], p['Ws']], axis=0)
        kqvs = _matmul_t(h, wall)
        k = kqvs[:, :HD]
        q = kqvs[:, HD:2 * HD]
        v = kqvs[:, 2 * HD:3 * HD]
        s = kqvs[:, 3 * HD:]
        kt = jnp.concatenate([k[:, :HH], k[:, HH:]], axis=0)
        qvt = jnp.concatenate(
            [jnp.concatenate([q[:, :HH], v[:, :HH]], axis=1),
             jnp.concatenate([q[:, HH:], v[:, HH:]], axis=1)],
            axis=0)
        aggf = _edge_sc(kt, qvt, src, dst, zeros_half)
        agg = (aggf.reshape(NCORE, NP, HH)[:, :NN]
               .transpose(1, 0, 2).reshape(NN, HD))
        if i != 2:
            h = _post_conv(agg, s, p['b'], params['ln_g'], params['ln_b'])
            hs.append(h)
        else:
            agg3, s3 = agg, s

    hjk = _jk(hs[0], hs[1], agg3, s3, convs[2]['b'],
              params['Wjk'], params['bjk'])

    wa = params['Wp1'][:, :HD]
    wb = params['Wp1'][:, HD:2 * HD]
    wab = jnp.concatenate([wa, wb], axis=0)
    ab = _matmul_t(hjk, wab)
    tab = jnp.concatenate([ab[:, :HD], ab[:, HD:]], axis=0)

    ts, td = target_edge_index[0], target_edge_index[1]
    pa, pb = _pair_sc(tab, ts, td)

    po = jnp.concatenate(
        [pitch_score, onset_score,
         jnp.zeros((ETN, 1), jnp.float32)], axis=1)
    wpo = jnp.concatenate(
        [params['Wp1'][:, 2 * HD:],
         jnp.zeros((HD, 1), jnp.float32)], axis=1)

    return _mlp(pa, pb, po, wpo,
                params['bp1'].reshape(1, HD),
                params['Wp2'],
                params['bp2'].reshape(1, HD // 2),
                params['Wp3'].reshape(1, HD // 2),
                params['bp3'].reshape(1, 1))
